# baseline (speedup 1.0000x reference)
import functools

import jax
import jax.numpy as jnp
import numpy as np
from jax import lax
from jax.experimental import pallas as pl
from jax.experimental.pallas import tpu as pltpu
from jax.experimental.pallas import tpu_sc as plsc


@functools.lru_cache(maxsize=None)
def _prefix_mats(n):
    gs = n // 128
    j = np.arange(n)[:, None]
    g = np.arange(128)[None, :]
    mg = ((j // gs) <= g).astype(np.float32)
    l2 = ((j % gs) <= g).astype(np.float32)
    return mg, l2


def _select_topk(s, t, r, mg, l2):
    R, N = s.shape
    gs = N // 128

    gt = s > t
    eq = s == t
    eqf = jnp.where(eq, 1.0, 0.0).astype(jnp.bfloat16)
    p = jnp.dot(eqf, mg, preferred_element_type=jnp.float32)
    gstar = jnp.sum(jnp.where(p < r, 1.0, 0.0), axis=1, keepdims=True)
    gcol = jax.lax.broadcasted_iota(jnp.int32, (R, 128), 1).astype(jnp.float32)
    before = jnp.sum(jnp.where(gcol == gstar - 1.0, p, 0.0), axis=1,
                     keepdims=True)
    r_in = r - before

    idx = jax.lax.broadcasted_iota(jnp.int32, (R, N), 1).astype(jnp.float32)
    gidx = jnp.floor(idx * (1.0 / gs))
    eqg = jnp.where(eq & (gidx == gstar), 1.0, 0.0).astype(jnp.bfloat16)
    q = jnp.dot(eqg, l2, preferred_element_type=jnp.float32)
    in_range = gcol < float(gs)
    ostar = jnp.sum(jnp.where(in_range & (q < r_in), 1.0, 0.0), axis=1,
                    keepdims=True)
    m = gstar * float(gs) + ostar
    return jnp.where(gt | (eq & (idx <= m)), 1.0, 0.0)


def _tc1_body(ks_ref, x_ref, wxh_ref, wxy_ref, sh_ref, y0_ref):
    x = x_ref[...]
    sh_ref[...] = jnp.dot(x, wxh_ref[...], preferred_element_type=jnp.float32)
    y0_ref[...] = jnp.dot(x, wxy_ref[...], preferred_element_type=jnp.float32)


def _tc2_body(ks_ref, sh_ref, kap_ref, y0_ref, why_ref, mg_ref, l2_ref,
              h_ref, d_ref, *, nh):
    s = sh_ref[...]
    kap = kap_ref[...]
    t = jnp.floor(kap * (1.0 / nh))
    cnt_gt = kap - t * float(nh)
    r = ks_ref[0].astype(jnp.float32) - cnt_gt
    h = _select_topk(s, t, r, mg_ref[...], l2_ref[...])
    h_ref[...] = h
    inh = jnp.dot(h.astype(jnp.bfloat16), why_ref[...],
                  preferred_element_type=jnp.float32)
    d_ref[...] = y0_ref[...] - inh


def _tc3_body(ks_ref, d_ref, kap_ref, mg_ref, l2_ref, y_ref, *, ny):
    d = d_ref[...]
    kap = kap_ref[...]
    t = jnp.floor(kap * (1.0 / ny))
    cnt_gt = kap - t * float(ny)
    r = ks_ref[1].astype(jnp.float32) - cnt_gt
    y_ref[...] = _select_topk(d, t, r, mg_ref[...], l2_ref[...])


def _sc_threshold(n, r_total, bins_words):
    nw = 32
    rpw = r_total // nw
    nchunks = n // 16
    mesh = plsc.VectorSubcoreMesh(core_axis_name="c", subcore_axis_name="s")

    @functools.partial(
        pl.kernel, mesh=mesh,
        out_type=jax.ShapeDtypeStruct((r_total,), jnp.float32),
        compiler_params=pltpu.CompilerParams(needs_layout_passes=False),
        scratch_types=[
            pltpu.VMEM((n,), jnp.float32),
            pltpu.VMEM((bins_words,), jnp.int32),
            pltpu.VMEM((16,), jnp.int32),
            pltpu.VMEM((rpw,), jnp.float32),
        ],
    )
    def body(s_hbm, k_hbm, out_hbm, rowbuf, bins, kv, outv):
        cid = lax.axis_index("c")
        sid = lax.axis_index("s")
        wid = sid * 2 + cid
        base = wid * rpw
        pltpu.sync_copy(k_hbm, kv)
        k = kv[...][0]
        lanes = lax.iota(jnp.int32, 16)
        zv = jnp.zeros((16,), jnp.int32)
        onev = jnp.ones((16,), jnp.int32)

        def row_loop(r_i, carry):
            row = base + r_i
            pltpu.sync_copy(s_hbm.at[row], rowbuf)

            def p1(j, c):
                vmx, vmn = c
                v = rowbuf[pl.ds(j * 16, 16)]
                return jnp.maximum(vmx, v), jnp.minimum(vmn, v)

            v0 = rowbuf[pl.ds(0, 16)]
            vmx, vmn = lax.fori_loop(1, nchunks, p1, (v0, v0))
            smx, _ = plsc.sort_key_val(vmx, vmx, descending=True)
            smn, _ = plsc.sort_key_val(vmn, vmn)
            mni = smn[0].astype(jnp.int32)
            mxi = smx[0].astype(jnp.int32)
            rng = mxi - mni + 1

            def zb(j, c):
                bins[pl.ds(j * 16, 16)] = zv
                return c

            lax.fori_loop(0, rng, zb, 0)

            def hist(j, c):
                v = rowbuf[pl.ds(j * 16, 16)]
                b = ((v.astype(jnp.int32) - mni) * 16) + lanes
                plsc.addupdate_scatter(bins, [b], onev)
                return c

            lax.fori_loop(0, nchunks, hist, 0)

            def sc_cond(c):
                cnt, b = c
                return cnt < k

            def sc_body(c):
                cnt, b = c
                bb = bins[pl.ds(b * 16, 16)]
                return cnt + jnp.sum(bb), b - 1

            cnt, bend = lax.while_loop(sc_cond, sc_body,
                                       (jnp.int32(0), rng - 1))
            tbin = bend + 1
            hb = bins[pl.ds(tbin * 16, 16)]
            cnt_gt = cnt - jnp.sum(hb)
            tval = (tbin + mni).astype(jnp.float32)
            kapv = jnp.full((16,), tval * float(n), jnp.float32) + \
                cnt_gt.astype(jnp.float32)
            plsc.store_scatter(outv, [jnp.full((16,), r_i, jnp.int32)],
                               kapv, mask=lanes == 0)
            return carry

        lax.fori_loop(0, rpw, row_loop, 0)
        pltpu.sync_copy(outv, out_hbm.at[pl.ds(base, rpw)])

    return body


def kernel(x, w_xy, w_xh, w_hy, kh, ky):
    B, NX = x.shape
    NY = w_xy.shape[1]
    NH = w_xh.shape[1]
    RB = 128

    xb = x.astype(jnp.bfloat16)
    wxy = w_xy.astype(jnp.bfloat16)
    wxh = w_xh.astype(jnp.bfloat16)
    why = w_hy.astype(jnp.bfloat16)
    ks = jnp.stack([jnp.asarray(kh, jnp.int32), jnp.asarray(ky, jnp.int32)])

    rows = lambda i, ks: (i, 0)
    full = lambda i, ks: (0, 0)

    sh, y0 = pl.pallas_call(
        _tc1_body,
        grid_spec=pltpu.PrefetchScalarGridSpec(
            num_scalar_prefetch=1,
            grid=(B // RB,),
            in_specs=[
                pl.BlockSpec((RB, NX), rows),
                pl.BlockSpec((NX, NH), full),
                pl.BlockSpec((NX, NY), full),
            ],
            out_specs=[
                pl.BlockSpec((RB, NH), rows),
                pl.BlockSpec((RB, NY), rows),
            ],
        ),
        out_shape=[
            jax.ShapeDtypeStruct((B, NH), jnp.float32),
            jax.ShapeDtypeStruct((B, NY), jnp.float32),
        ],
        compiler_params=pltpu.CompilerParams(
            dimension_semantics=("arbitrary",),
        ),
    )(ks, xb, wxh, wxy)

    k16h = jnp.full((16,), jnp.asarray(kh, jnp.int32))
    kap_h = _sc_threshold(NH, B, (NX + 1) * 16)(sh, k16h)

    mgh_np, l2h_np = _prefix_mats(NH)
    mgy_np, l2y_np = _prefix_mats(NY)
    mgh = jnp.asarray(mgh_np, jnp.bfloat16)
    l2h = jnp.asarray(l2h_np, jnp.bfloat16)
    mgy = jnp.asarray(mgy_np, jnp.bfloat16)
    l2y = jnp.asarray(l2y_np, jnp.bfloat16)

    h, d = pl.pallas_call(
        functools.partial(_tc2_body, nh=NH),
        grid_spec=pltpu.PrefetchScalarGridSpec(
            num_scalar_prefetch=1,
            grid=(B // RB,),
            in_specs=[
                pl.BlockSpec((RB, NH), rows),
                pl.BlockSpec((RB, 1), rows),
                pl.BlockSpec((RB, NY), rows),
                pl.BlockSpec((NH, NY), full),
                pl.BlockSpec((NH, 128), full),
                pl.BlockSpec((NH, 128), full),
            ],
            out_specs=[
                pl.BlockSpec((RB, NH), rows),
                pl.BlockSpec((RB, NY), rows),
            ],
        ),
        out_shape=[
            jax.ShapeDtypeStruct((B, NH), jnp.float32),
            jax.ShapeDtypeStruct((B, NY), jnp.float32),
        ],
        compiler_params=pltpu.CompilerParams(
            dimension_semantics=("arbitrary",),
        ),
    )(ks, sh, kap_h.reshape(B, 1), y0, why, mgh, l2h)

    k16y = jnp.full((16,), jnp.asarray(ky, jnp.int32))
    kap_y = _sc_threshold(NY, B, (NX + NH + 1) * 16)(d, k16y)

    (y,) = pl.pallas_call(
        functools.partial(_tc3_body, ny=NY),
        grid_spec=pltpu.PrefetchScalarGridSpec(
            num_scalar_prefetch=1,
            grid=(B // RB,),
            in_specs=[
                pl.BlockSpec((RB, NY), rows),
                pl.BlockSpec((RB, 1), rows),
                pl.BlockSpec((NY, 128), full),
                pl.BlockSpec((NY, 128), full),
            ],
            out_specs=[pl.BlockSpec((RB, NY), rows)],
        ),
        out_shape=[jax.ShapeDtypeStruct((B, NY), jnp.float32)],
        compiler_params=pltpu.CompilerParams(
            dimension_semantics=("arbitrary",),
        ),
    )(ks, d, kap_y.reshape(B, 1), mgy, l2y)

    return h, y
